# baseline (speedup 1.0000x reference)
import jax
import jax.numpy as jnp
from jax.experimental import pallas as pl
from jax.experimental.pallas import tpu as pltpu

_LN2 = 0.6931471805599453
_FOCAL_ALPHA = 0.25
_SIGMA_SQ = 9.0
_POS_THRESH = 0.5
_NEG_THRESH = 0.4
_EPS = 1e-4
_TA = 12000


def _retina_block(lim_ref, ann_ref, anc_ref, reg_ref, cls_ref,
                  loss_out, npos_out):
    i = pl.program_id(1)

    bb = pl.program_id(0)
    m = ann_ref.shape[1]
    tr = anc_ref.shape[2]

    at = jnp.transpose(anc_ref[0, 0], (1, 0)).reshape(32, 4, tr)
    ax1 = at[:, 0, :]
    ay1 = at[:, 1, :]
    ax2 = at[:, 2, :]
    ay2 = at[:, 3, :]
    rt = jnp.transpose(reg_ref[0, 0], (1, 0)).reshape(32, 4, tr)

    area_a = (ax2 - ax1) * (ay2 - ay1)
    max_iou = jnp.full_like(ax1, -1.0)
    gx1 = jnp.zeros_like(ax1)
    gy1 = jnp.zeros_like(ax1)
    gx2 = jnp.zeros_like(ax1)
    gy2 = jnp.zeros_like(ax1)
    gclf = jnp.zeros_like(ax1)
    for mm in range(m):
        x1 = ann_ref[bb, mm, 0]
        y1 = ann_ref[bb, mm, 1]
        x2 = ann_ref[bb, mm, 2]
        y2 = ann_ref[bb, mm, 3]
        cl = ann_ref[bb, mm, 4]
        iwm = jnp.maximum(jnp.minimum(ax2, x2) - jnp.maximum(ax1, x1), 0.0)
        ihm = jnp.maximum(jnp.minimum(ay2, y2) - jnp.maximum(ay1, y1), 0.0)
        interm = iwm * ihm
        ab = (x2 - x1) * (y2 - y1)
        ioum = interm / jnp.maximum(area_a + (ab - interm), 1e-8)
        upd = ioum > max_iou
        max_iou = jnp.where(upd, ioum, max_iou)
        gx1 = jnp.where(upd, x1, gx1)
        gy1 = jnp.where(upd, y1, gy1)
        gx2 = jnp.where(upd, x2, gx2)
        gy2 = jnp.where(upd, y2, gy2)
        gclf = jnp.where(upd, cl, gclf)
    gcl = gclf.astype(jnp.int32)

    hf = lim_ref[0]
    wf = lim_ref[1]
    cx = (ax1 + ax2) * 0.5
    cy = (ay1 + ay2) * 0.5
    inside = (cx < wf) & (cy < hf)
    pos_raw = max_iou >= _POS_THRESH
    pos = pos_raw & inside
    valid = (pos_raw | (max_iou <= _NEG_THRESH)) & inside
    posf = pos.astype(jnp.float32)
    validf = valid.astype(jnp.float32)

    aw = ax2 - ax1
    ah = ay2 - ay1
    sw = 5.0 / aw
    sh = 5.0 / ah
    t0 = (gx1 - ax1) * sw
    t1 = (gy1 - ay1) * sh
    t2 = (gx2 - ax2) * sw
    t3 = (gy2 - ay2) * sh
    reg_sum = None
    for c, tc in enumerate((t0, t1, t2, t3)):
        diff = jnp.abs(rt[:, c, :] - tc)
        sl1 = jnp.where(diff < 1.0 / _SIGMA_SQ,
                        (0.5 * _SIGMA_SQ) * diff * diff,
                        diff - 0.5 / _SIGMA_SQ)
        reg_sum = sl1 if reg_sum is None else reg_sum + sl1
    reg_plane = reg_sum * posf

    pt = jnp.transpose(cls_ref[0, 0], (1, 0)).reshape(32, 80, tr)
    p = jnp.clip(pt, _EPS, 1.0 - _EPS)
    tcl = jnp.where(pos, gcl, -1)[:, None, :]
    csub = jax.lax.broadcasted_iota(jnp.int32, p.shape, 1)
    isp = csub == tcl
    s = jnp.where(isp, p, 1.0 - p)
    alpha_l = jnp.where(isp, -_LN2 * _FOCAL_ALPHA, -_LN2 * (1.0 - _FOCAL_ALPHA))
    oms = 1.0 - s
    cls_elem = alpha_l * (oms * oms) * jnp.log2(s)
    cls_plane = jnp.sum(cls_elem, axis=1) * validf

    loss_plane = cls_plane + reg_plane

    @pl.when(i == 0)
    def _():
        loss_out[0, 0] = jnp.zeros_like(loss_out)[0, 0]
        npos_out[0, 0] = jnp.zeros_like(npos_out)[0, 0]

    loss_out[0, 0] = loss_out[0, 0] + loss_plane
    npos_out[0, 0] = npos_out[0, 0] + posf


def kernel(output_regression, output_classification, batch_annotations,
           anchors, image_shape):
    B, A, C = output_classification.shape
    M = batch_annotations.shape[1]
    ta = _TA if A % _TA == 0 else A
    nb = A // ta
    tr = ta // 32

    anc_v = anchors.reshape(B, nb, ta * 4 // 128, 128)
    reg_v = output_regression.reshape(B, nb, ta * 4 // 128, 128)
    cls_v = output_classification.reshape(B, nb, ta // 32, 32 * C)
    lims = image_shape.astype(jnp.float32)

    loss_p, np_p = pl.pallas_call(
        _retina_block,
        grid=(B, nb),
        in_specs=[
            pl.BlockSpec(memory_space=pltpu.SMEM),
            pl.BlockSpec(memory_space=pltpu.SMEM),
            pl.BlockSpec((1, 1, ta * 4 // 128, 128), lambda b, i: (b, i, 0, 0)),
            pl.BlockSpec((1, 1, ta * 4 // 128, 128), lambda b, i: (b, i, 0, 0)),
            pl.BlockSpec((1, 1, ta // 32, 32 * C), lambda b, i: (b, i, 0, 0)),
        ],
        out_specs=[
            pl.BlockSpec((1, 1, 32, tr), lambda b, i: (b, 0, 0, 0)),
            pl.BlockSpec((1, 1, 32, tr), lambda b, i: (b, 0, 0, 0)),
        ],
        out_shape=[
            jax.ShapeDtypeStruct((B, 1, 32, tr), jnp.float32),
            jax.ShapeDtypeStruct((B, 1, 32, tr), jnp.float32),
        ],
        compiler_params=pltpu.CompilerParams(
            dimension_semantics=("parallel", "arbitrary"),
        ),
    )(lims, batch_annotations, anc_v, reg_v, cls_v)

    npos = jnp.sum(np_p)
    norm = jnp.maximum(npos, 1.0)
    return jnp.sum(loss_p) / norm


# speedup vs baseline: 2.3693x; 2.3693x over previous
import jax
import jax.numpy as jnp
from jax.experimental import pallas as pl
from jax.experimental.pallas import tpu as pltpu

_LN2 = 0.6931471805599453
_FOCAL_ALPHA = 0.25
_SIGMA_SQ = 9.0
_POS_THRESH = 0.5
_NEG_THRESH = 0.4
_EPS = 1e-4
_TA = 3750


def _retina_block(lim_ref, ann_ref, anc_ref, reg_ref, cls_ref,
                  loss_out, npos_out):
    i = pl.program_id(1)
    nb = pl.num_programs(1)

    anc = anc_ref[0, 0]
    ann = ann_ref[0]
    m = ann.shape[0]

    ax1 = anc[0:1, :]
    ay1 = anc[1:2, :]
    ax2 = anc[2:3, :]
    ay2 = anc[3:4, :]
    bx1 = ann[:, 0:1]
    by1 = ann[:, 1:2]
    bx2 = ann[:, 2:3]
    by2 = ann[:, 3:4]
    bcl = ann[:, 4:5]

    iw = jnp.maximum(jnp.minimum(ax2, bx2) - jnp.maximum(ax1, bx1), 0.0)
    ih = jnp.maximum(jnp.minimum(ay2, by2) - jnp.maximum(ay1, by1), 0.0)
    inter = iw * ih
    area_a = (ax2 - ax1) * (ay2 - ay1)
    area_b = (bx2 - bx1) * (by2 - by1)
    iou = inter / jnp.maximum(area_a + area_b - inter, 1e-8)

    max_iou = jnp.max(iou, axis=0, keepdims=True)
    sub = jax.lax.broadcasted_iota(jnp.int32, iou.shape, 0)
    best = jnp.min(jnp.where(iou == max_iou, sub, m), axis=0, keepdims=True)
    oh = (sub == best).astype(jnp.float32)
    gx1 = jnp.sum(oh * bx1, axis=0, keepdims=True)
    gy1 = jnp.sum(oh * by1, axis=0, keepdims=True)
    gx2 = jnp.sum(oh * bx2, axis=0, keepdims=True)
    gy2 = jnp.sum(oh * by2, axis=0, keepdims=True)
    gcl = jnp.sum(oh * bcl, axis=0, keepdims=True).astype(jnp.int32)

    hf = lim_ref[0]
    wf = lim_ref[1]
    cx = (ax1 + ax2) * 0.5
    cy = (ay1 + ay2) * 0.5
    inside = (cx < wf) & (cy < hf)
    pos_raw = max_iou >= _POS_THRESH
    pos = pos_raw & inside
    valid = (pos_raw | (max_iou <= _NEG_THRESH)) & inside
    posf = pos.astype(jnp.float32)
    validf = valid.astype(jnp.float32)

    aw = ax2 - ax1
    ah = ay2 - ay1
    sw = 5.0 / aw
    sh = 5.0 / ah
    t = jnp.concatenate([(gx1 - ax1) * sw, (gy1 - ay1) * sh,
                         (gx2 - ax2) * sw, (gy2 - ay2) * sh], axis=0)
    diff = jnp.abs(reg_ref[0, 0] - t)
    sl1 = jnp.where(diff < 1.0 / _SIGMA_SQ,
                    (0.5 * _SIGMA_SQ) * diff * diff,
                    diff - 0.5 / _SIGMA_SQ)
    reg_row = jnp.sum(sl1, axis=0, keepdims=True) * posf

    pt = jnp.transpose(cls_ref[0, 0], (1, 0))
    p = jnp.clip(pt, _EPS, 1.0 - _EPS)
    csub = jax.lax.broadcasted_iota(jnp.int32, p.shape, 0)
    isp = (csub == gcl) & pos
    s = jnp.where(isp, p, 1.0 - p)
    alpha_l = jnp.where(isp, -_LN2 * _FOCAL_ALPHA, -_LN2 * (1.0 - _FOCAL_ALPHA))
    oms = 1.0 - s
    cls_elem = alpha_l * (oms * oms) * jnp.log2(s)
    cls_row = jnp.sum(cls_elem, axis=0, keepdims=True) * validf

    loss_row = cls_row + reg_row

    @pl.when(i == 0)
    def _():
        loss_out[0] = jnp.zeros_like(loss_out)[0]
        npos_out[0] = jnp.zeros_like(npos_out)[0]

    loss_out[0] = loss_out[0] + loss_row
    npos_out[0] = npos_out[0] + posf


def kernel(output_regression, output_classification, batch_annotations,
           anchors, image_shape):
    B, A, C = output_classification.shape
    M = batch_annotations.shape[1]
    ta = _TA if A % _TA == 0 else A
    nb = A // ta

    anc_r = anchors.transpose(0, 2, 1).reshape(B, 4, nb, ta).transpose(0, 2, 1, 3)
    reg_r = output_regression.transpose(0, 2, 1).reshape(B, 4, nb, ta).transpose(0, 2, 1, 3)
    cls_r = output_classification.reshape(B, nb, ta, C)
    lims = image_shape.astype(jnp.float32)

    loss_p, np_p = pl.pallas_call(
        _retina_block,
        grid=(B, nb),
        in_specs=[
            pl.BlockSpec(memory_space=pltpu.SMEM),
            pl.BlockSpec((1, M, 5), lambda b, i: (b, 0, 0)),
            pl.BlockSpec((1, 1, 4, ta), lambda b, i: (b, i, 0, 0)),
            pl.BlockSpec((1, 1, 4, ta), lambda b, i: (b, i, 0, 0)),
            pl.BlockSpec((1, 1, ta, C), lambda b, i: (b, i, 0, 0)),
        ],
        out_specs=[
            pl.BlockSpec((1, 1, ta), lambda b, i: (b, 0, 0)),
            pl.BlockSpec((1, 1, ta), lambda b, i: (b, 0, 0)),
        ],
        out_shape=[
            jax.ShapeDtypeStruct((B, 1, ta), jnp.float32),
            jax.ShapeDtypeStruct((B, 1, ta), jnp.float32),
        ],
        compiler_params=pltpu.CompilerParams(
            dimension_semantics=("parallel", "arbitrary"),
        ),
    )(lims, batch_annotations, anc_r, reg_r, cls_r)

    npos = jnp.sum(np_p)
    norm = jnp.maximum(npos, 1.0)
    return jnp.sum(loss_p) / norm
